# baseline (speedup 1.0000x reference)
import jax
import jax.numpy as jnp
from jax.experimental import pallas as pl
from jax.experimental.pallas import tpu as pltpu

E = 8
TOP_K = 2
N_GROUPS = 4
GS = E // N_GROUPS
NE = E + 1
NH = 4


def _routing(scores, T):
    lane = jax.lax.broadcasted_iota(jnp.int32, (T, E), 1)
    grp = lane // GS
    partner = jnp.where(lane % 2 == 0, jnp.roll(scores, -1, axis=1),
                        jnp.roll(scores, 1, axis=1))
    gsum = scores + partner
    g1 = jnp.argmax(gsum, axis=-1, keepdims=True) // GS
    gsum2 = jnp.where(grp == g1, -jnp.inf, gsum)
    g2 = jnp.argmax(gsum2, axis=-1, keepdims=True) // GS
    ms = jnp.where((grp == g1) | (grp == g2), scores, 0.0)
    i1 = jnp.argmax(ms, axis=-1, keepdims=True)
    v1 = jnp.max(ms, axis=-1, keepdims=True)
    ms2 = jnp.where(lane == i1, -jnp.inf, ms)
    i2 = jnp.argmax(ms2, axis=-1, keepdims=True)
    v2 = jnp.max(ms2, axis=-1, keepdims=True)
    den = v1 + v2 + 1e-20
    lane16 = jax.lax.broadcasted_iota(jnp.int32, (T, 16), 1)
    tw = jnp.where(lane16 == i1, v1 / den, 0.0)
    tw = jnp.where(lane16 == i2, v2 / den, tw)
    tw = jnp.where(lane16 == E, 1.0, tw)
    return tw


def _moe_dense_kernel(x_ref, sc_ref, w1_ref, w2_ref, out_ref, tokw_ref):
    e = pl.program_id(0)
    h = pl.program_id(1)
    T = x_ref.shape[0]

    @pl.when((e == 0) & (h == 0))
    def _():
        tokw_ref[...] = _routing(sc_ref[...], T)
        out_ref[...] = jnp.zeros_like(out_ref)

    x = x_ref[...]
    hb = jax.nn.silu(jnp.dot(x, w1_ref[0], preferred_element_type=jnp.float32))
    part = jnp.dot(hb, w2_ref[0], preferred_element_type=jnp.float32)
    lane16 = jax.lax.broadcasted_iota(jnp.int32, (T, 16), 1)
    ew = jnp.sum(jnp.where(lane16 == e, tokw_ref[...], 0.0), axis=1, keepdims=True)
    out_ref[...] += part * ew


def kernel(x, gate_w, w1, w2, ws1, ws2, bias):
    B, T, D = x.shape
    H = w1.shape[2]
    xf = x.reshape(T, D)
    W1 = jnp.concatenate([w1, ws1[None]], axis=0)
    W2 = jnp.concatenate([w2, ws2[None]], axis=0)
    scores = jax.nn.sigmoid(jnp.dot(xf, gate_w.T)) + bias[None, :]
    Hb = H // NH

    out = pl.pallas_call(
        _moe_dense_kernel,
        grid=(NE, NH),
        in_specs=[
            pl.BlockSpec((T, D), lambda e, h: (0, 0)),
            pl.BlockSpec((T, E), lambda e, h: (0, 0)),
            pl.BlockSpec((1, D, Hb), lambda e, h: (e, 0, h)),
            pl.BlockSpec((1, Hb, D), lambda e, h: (e, h, 0)),
        ],
        out_specs=pl.BlockSpec((T, D), lambda e, h: (0, 0)),
        out_shape=jax.ShapeDtypeStruct((T, D), jnp.float32),
        scratch_shapes=[pltpu.VMEM((T, 16), jnp.float32)],
        compiler_params=pltpu.CompilerParams(
            dimension_semantics=("arbitrary", "arbitrary"),
        ),
    )(xf, scores, W1, W2)
    return out.reshape(B, T, D)
